# baseline (device time: 60003 ns/iter reference)
import jax
import jax.numpy as jnp
from jax import lax
from jax.experimental import pallas as pl
from jax.experimental.pallas import tpu as pltpu

N_DEV = 16
N_BUF = 5


def kernel(x, w_mat):
    k_total, k_blk = x.shape
    _, n_total = w_mat.shape
    m_per = k_total // N_DEV

    def body(x_ref, w_ref, out_ref, xbf_ref, xrow_ref, wbuf_ref,
             send_sems, recv_sems, load_sems):
        my = lax.axis_index("i")

        def kb_of(j):
            return lax.rem(my - j + N_DEV, N_DEV)

        half = m_per // 2

        def w_loads(j, slot):
            base = kb_of(j) * m_per
            return [
                pltpu.make_async_copy(
                    w_ref.at[pl.ds(base + h * half, half), :],
                    wbuf_ref.at[slot, pl.ds(h * half, half), :],
                    load_sems.at[slot, h],
                )
                for h in range(2)
            ]

        for j in range(N_BUF):
            for c in w_loads(j, j):
                c.start()

        xbf_ref[:, :] = x_ref[:, :].astype(jnp.bfloat16)

        barrier = pltpu.get_barrier_semaphore()
        for j in range(1, N_DEV):
            peer = lax.rem(my + j, N_DEV)
            pl.semaphore_signal(
                barrier, inc=1,
                device_id=(peer,), device_id_type=pl.DeviceIdType.MESH,
            )
        pl.semaphore_wait(barrier, N_DEV - 1)

        sends = []
        for j in range(1, N_DEV):
            dst = lax.rem(my + j, N_DEV)
            rdma = pltpu.make_async_remote_copy(
                src_ref=xbf_ref.at[pl.ds(dst * m_per, m_per), :],
                dst_ref=xrow_ref.at[:, pl.ds(my * k_blk, k_blk)],
                send_sem=send_sems.at[j],
                recv_sem=recv_sems.at[j],
                device_id=(dst,),
                device_id_type=pl.DeviceIdType.MESH,
            )
            rdma.start()
            sends.append(rdma)

        xrow_ref[:, pl.ds(my * k_blk, k_blk)] = xbf_ref[
            pl.ds(my * m_per, m_per), :
        ]

        for j in range(N_DEV):
            slot = j % N_BUF
            for c in w_loads(j, slot):
                c.wait()
            kb = kb_of(j)
            if j > 0:
                recv = pltpu.make_async_remote_copy(
                    src_ref=xbf_ref.at[pl.ds(0, m_per), :],
                    dst_ref=xrow_ref.at[:, pl.ds(kb * k_blk, k_blk)],
                    send_sem=send_sems.at[j],
                    recv_sem=recv_sems.at[j],
                    device_id=(kb,),
                    device_id_type=pl.DeviceIdType.MESH,
                )
                recv.wait_recv()
            partial = jnp.dot(
                xrow_ref[:, pl.ds(kb * k_blk, k_blk)],
                wbuf_ref[slot].astype(jnp.bfloat16),
                preferred_element_type=jnp.float32,
            )
            if j == 0:
                out_ref[:, :] = partial
            elif j < N_DEV - 1:
                out_ref[:, :] = out_ref[:, :] + partial
            else:
                y = out_ref[:, :] + partial
                out_ref[:, :] = y * jax.nn.sigmoid(y)
            nxt = j + N_BUF
            if nxt < N_DEV:
                for c in w_loads(nxt, slot):
                    c.start()

        for rdma in sends:
            rdma.wait_send()

    return pl.pallas_call(
        body,
        in_specs=[
            pl.BlockSpec(memory_space=pltpu.VMEM),
            pl.BlockSpec(memory_space=pl.ANY),
        ],
        out_specs=pl.BlockSpec(memory_space=pltpu.VMEM),
        out_shape=jax.ShapeDtypeStruct((m_per, n_total), jnp.float32),
        scratch_shapes=[
            pltpu.VMEM((k_total, k_blk), jnp.bfloat16),
            pltpu.VMEM((m_per, k_total), jnp.bfloat16),
            pltpu.VMEM((N_BUF, m_per, n_total), jnp.float32),
            pltpu.SemaphoreType.DMA((N_DEV,)),
            pltpu.SemaphoreType.DMA((N_DEV,)),
            pltpu.SemaphoreType.DMA((N_BUF, 2)),
        ],
        compiler_params=pltpu.CompilerParams(
            collective_id=0,
            vmem_limit_bytes=100 * 1024 * 1024,
        ),
    )(x, w_mat)


# device time: 58616 ns/iter; 1.0237x vs baseline; 1.0237x over previous
import jax
import jax.numpy as jnp
from jax import lax
from jax.experimental import pallas as pl
from jax.experimental.pallas import tpu as pltpu

N_DEV = 16
N_BUF = 4


def kernel(x, w_mat):
    k_total, k_blk = x.shape
    _, n_total = w_mat.shape
    m_per = k_total // N_DEV

    def body(x_ref, w_ref, out_ref, xbf_ref, xrow_ref, wbuf_ref,
             send_sems, recv_sems, load_sems):
        my = lax.axis_index("i")

        def kb_of(j):
            return lax.rem(my - j + N_DEV, N_DEV)

        def w_load(j, slot):
            return pltpu.make_async_copy(
                w_ref.at[pl.ds(kb_of(j) * m_per, m_per), :],
                wbuf_ref.at[slot],
                load_sems.at[slot],
            )

        for j in range(N_BUF):
            w_load(j, j).start()

        xbf_ref[:, :] = x_ref[:, :].astype(jnp.bfloat16)

        barrier = pltpu.get_barrier_semaphore()
        for j in range(1, N_DEV):
            peer = lax.rem(my + j, N_DEV)
            pl.semaphore_signal(
                barrier, inc=1,
                device_id=(peer,), device_id_type=pl.DeviceIdType.MESH,
            )
        pl.semaphore_wait(barrier, N_DEV - 1)

        sends = []
        for j in range(1, N_DEV):
            dst = lax.rem(my + j, N_DEV)
            rdma = pltpu.make_async_remote_copy(
                src_ref=xbf_ref.at[pl.ds(dst * m_per, m_per), :],
                dst_ref=xrow_ref.at[:, pl.ds(my * k_blk, k_blk)],
                send_sem=send_sems.at[j],
                recv_sem=recv_sems.at[j],
                device_id=(dst,),
                device_id_type=pl.DeviceIdType.MESH,
            )
            rdma.start()
            sends.append(rdma)

        xrow_ref[:, pl.ds(my * k_blk, k_blk)] = xbf_ref[
            pl.ds(my * m_per, m_per), :
        ]

        for j in range(N_DEV):
            slot = j % N_BUF
            w_load(j, slot).wait()
            kb = kb_of(j)
            if j > 0:
                recv = pltpu.make_async_remote_copy(
                    src_ref=xbf_ref.at[pl.ds(0, m_per), :],
                    dst_ref=xrow_ref.at[:, pl.ds(kb * k_blk, k_blk)],
                    send_sem=send_sems.at[j],
                    recv_sem=recv_sems.at[j],
                    device_id=(kb,),
                    device_id_type=pl.DeviceIdType.MESH,
                )
                recv.wait_recv()
            partial = jnp.dot(
                xrow_ref[:, pl.ds(kb * k_blk, k_blk)],
                wbuf_ref[slot].astype(jnp.bfloat16),
                preferred_element_type=jnp.float32,
            )
            if j == 0:
                out_ref[:, :] = partial
            elif j < N_DEV - 1:
                out_ref[:, :] = out_ref[:, :] + partial
            else:
                y = out_ref[:, :] + partial
                out_ref[:, :] = y * jax.nn.sigmoid(y)
            nxt = j + N_BUF
            if nxt < N_DEV:
                w_load(nxt, slot).start()

        for rdma in sends:
            rdma.wait_send()

    return pl.pallas_call(
        body,
        in_specs=[
            pl.BlockSpec(memory_space=pltpu.VMEM),
            pl.BlockSpec(memory_space=pl.ANY),
        ],
        out_specs=pl.BlockSpec(memory_space=pltpu.VMEM),
        out_shape=jax.ShapeDtypeStruct((m_per, n_total), jnp.float32),
        scratch_shapes=[
            pltpu.VMEM((k_total, k_blk), jnp.bfloat16),
            pltpu.VMEM((m_per, k_total), jnp.bfloat16),
            pltpu.VMEM((N_BUF, m_per, n_total), jnp.float32),
            pltpu.SemaphoreType.DMA((N_DEV,)),
            pltpu.SemaphoreType.DMA((N_DEV,)),
            pltpu.SemaphoreType.DMA((N_BUF,)),
        ],
        compiler_params=pltpu.CompilerParams(
            collective_id=0,
            vmem_limit_bytes=100 * 1024 * 1024,
        ),
    )(x, w_mat)


# device time: 56873 ns/iter; 1.0550x vs baseline; 1.0306x over previous
import jax
import jax.numpy as jnp
from jax import lax
from jax.experimental import pallas as pl
from jax.experimental.pallas import tpu as pltpu

N_DEV = 16
N_BUF = 4


def kernel(x, w_mat):
    k_total, k_blk = x.shape
    _, n_total = w_mat.shape
    m_per = k_total // N_DEV

    def body(x_ref, w_ref, out_ref, xbf_ref, xrow_ref, wbuf_ref, acc_ref,
             send_sems, recv_sems, load_sems, out_sems):
        my = lax.axis_index("i")

        def kb_of(j):
            return lax.rem(my - j + N_DEV, N_DEV)

        def w_load(j, slot):
            return pltpu.make_async_copy(
                w_ref.at[pl.ds(kb_of(j) * m_per, m_per), :],
                wbuf_ref.at[slot],
                load_sems.at[slot],
            )

        for j in range(N_BUF):
            w_load(j, j).start()

        xbf_ref[:, :] = x_ref[:, :].astype(jnp.bfloat16)

        barrier = pltpu.get_barrier_semaphore()
        for j in range(1, N_DEV):
            peer = lax.rem(my + j, N_DEV)
            pl.semaphore_signal(
                barrier, inc=1,
                device_id=(peer,), device_id_type=pl.DeviceIdType.MESH,
            )
        pl.semaphore_wait(barrier, N_DEV - 1)

        sends = []
        for j in range(1, N_DEV):
            dst = lax.rem(my + j, N_DEV)
            rdma = pltpu.make_async_remote_copy(
                src_ref=xbf_ref.at[pl.ds(dst * m_per, m_per), :],
                dst_ref=xrow_ref.at[:, pl.ds(my * k_blk, k_blk)],
                send_sem=send_sems.at[j],
                recv_sem=recv_sems.at[j],
                device_id=(dst,),
                device_id_type=pl.DeviceIdType.MESH,
            )
            rdma.start()
            sends.append(rdma)

        xrow_ref[:, pl.ds(my * k_blk, k_blk)] = xbf_ref[
            pl.ds(my * m_per, m_per), :
        ]

        out_copies = []
        for j in range(N_DEV):
            slot = j % N_BUF
            w_load(j, slot).wait()
            kb = kb_of(j)
            if j > 0:
                recv = pltpu.make_async_remote_copy(
                    src_ref=xbf_ref.at[pl.ds(0, m_per), :],
                    dst_ref=xrow_ref.at[:, pl.ds(kb * k_blk, k_blk)],
                    send_sem=send_sems.at[j],
                    recv_sem=recv_sems.at[j],
                    device_id=(kb,),
                    device_id_type=pl.DeviceIdType.MESH,
                )
                recv.wait_recv()
            if j < N_DEV - 1:
                partial = jnp.dot(
                    xrow_ref[:, pl.ds(kb * k_blk, k_blk)],
                    wbuf_ref[slot].astype(jnp.bfloat16),
                    preferred_element_type=jnp.float32,
                )
                if j == 0:
                    acc_ref[:, :] = partial
                else:
                    acc_ref[:, :] = acc_ref[:, :] + partial
            else:
                n_q = n_total // 4
                for q in range(4):
                    qs = pl.ds(q * n_q, n_q)
                    p_q = jnp.dot(
                        xrow_ref[:, pl.ds(kb * k_blk, k_blk)],
                        wbuf_ref[slot, :, qs].astype(jnp.bfloat16),
                        preferred_element_type=jnp.float32,
                    )
                    y = acc_ref[:, qs] + p_q
                    acc_ref[:, qs] = y * jax.nn.sigmoid(y)
                    out_copies.append(
                        pltpu.make_async_copy(
                            acc_ref.at[:, qs], out_ref.at[:, qs],
                            out_sems.at[q],
                        )
                    )
                    out_copies[-1].start()
            nxt = j + N_BUF
            if nxt < N_DEV:
                w_load(nxt, slot).start()

        for rdma in sends:
            rdma.wait_send()
        for c in out_copies:
            c.wait()

    return pl.pallas_call(
        body,
        in_specs=[
            pl.BlockSpec(memory_space=pltpu.VMEM),
            pl.BlockSpec(memory_space=pl.ANY),
        ],
        out_specs=pl.BlockSpec(memory_space=pl.ANY),
        out_shape=jax.ShapeDtypeStruct((m_per, n_total), jnp.float32),
        scratch_shapes=[
            pltpu.VMEM((k_total, k_blk), jnp.bfloat16),
            pltpu.VMEM((m_per, k_total), jnp.bfloat16),
            pltpu.VMEM((N_BUF, m_per, n_total), jnp.float32),
            pltpu.VMEM((m_per, n_total), jnp.float32),
            pltpu.SemaphoreType.DMA((N_DEV,)),
            pltpu.SemaphoreType.DMA((N_DEV,)),
            pltpu.SemaphoreType.DMA((N_BUF,)),
            pltpu.SemaphoreType.DMA((4,)),
        ],
        compiler_params=pltpu.CompilerParams(
            collective_id=0,
            vmem_limit_bytes=100 * 1024 * 1024,
        ),
    )(x, w_mat)
